# Initial kernel scaffold; baseline (speedup 1.0000x reference)
import jax
import jax.numpy as jnp
from jax.experimental import pallas as pl


def kernel(x, edge_index, W, b):
    raise NotImplementedError("write your pallas kernel here")


# speedup vs baseline: 23.8168x; 23.8168x over previous
import functools

import jax
import jax.numpy as jnp
from jax import lax
from jax.experimental import pallas as pl
from jax.experimental.pallas import tpu as pltpu
from jax.experimental.pallas import tpu_sc as plsc

N = 10000
E = 320000
D = 128
NC = 2
NS = 16
NW = NC * NS
CHUNK = 128
CPT = 79
EP = NW * CPT * CHUNK
NP = 10240
RPS = NP // NS

_mesh = plsc.VectorSubcoreMesh(core_axis_name="c", subcore_axis_name="s")


@functools.partial(
    pl.kernel,
    mesh=_mesh,
    out_type=jax.ShapeDtypeStruct((NC, NP, D), jnp.float32),
    scratch_types=[
        pltpu.VMEM((CPT, CHUNK), jnp.int32),
        pltpu.VMEM((CHUNK, D), jnp.float32),
        pltpu.VMEM_SHARED((NP, D), jnp.float32),
    ],
)
def _sc_degree(col_hbm, zeros_hbm, ones_hbm, deg_out, idx_v, ones_v, deg_sh):
    cid = lax.axis_index("c")
    sid = lax.axis_index("s")
    wid = cid * NS + sid
    pltpu.sync_copy(zeros_hbm.at[pl.ds(sid * RPS, RPS)],
                    deg_sh.at[pl.ds(sid * RPS, RPS)])
    pltpu.sync_copy(ones_hbm, ones_v)
    pltpu.sync_copy(col_hbm.at[wid], idx_v)
    plsc.subcore_barrier()

    @pl.loop(0, CPT)
    def _(j):
        pltpu.sync_copy(ones_v, deg_sh.at[idx_v.at[j]], add=True)

    plsc.subcore_barrier()
    pltpu.sync_copy(deg_sh.at[pl.ds(sid * RPS, RPS)],
                    deg_out.at[cid].at[pl.ds(sid * RPS, RPS)])


@functools.partial(
    pl.kernel,
    mesh=_mesh,
    out_type=jax.ShapeDtypeStruct((NC, NP, D), jnp.float32),
    scratch_types=[
        pltpu.VMEM((CPT, CHUNK), jnp.int32),
        pltpu.VMEM((CPT, CHUNK), jnp.int32),
        pltpu.VMEM((CHUNK, D), jnp.float32),
        pltpu.VMEM_SHARED((NP, D), jnp.float32),
        pltpu.SemaphoreType.DMA,
    ],
)
def _sc_aggregate(row_hbm, col_hbm, y_hbm, zeros_hbm, acc_out,
                  ridx_v, cidx_v, gbuf, acc_sh, sem):
    cid = lax.axis_index("c")
    sid = lax.axis_index("s")
    wid = cid * NS + sid
    pltpu.sync_copy(zeros_hbm.at[pl.ds(sid * RPS, RPS)],
                    acc_sh.at[pl.ds(sid * RPS, RPS)])
    pltpu.sync_copy(row_hbm.at[wid], ridx_v)
    pltpu.sync_copy(col_hbm.at[wid], cidx_v)
    plsc.subcore_barrier()

    @pl.loop(0, CPT)
    def _(j):
        pltpu.async_copy(y_hbm.at[ridx_v.at[j]], gbuf, sem).wait()
        pltpu.sync_copy(gbuf, acc_sh.at[cidx_v.at[j]], add=True)

    plsc.subcore_barrier()
    pltpu.sync_copy(acc_sh.at[pl.ds(sid * RPS, RPS)],
                    acc_out.at[cid].at[pl.ds(sid * RPS, RPS)])


_BLK = 512
_GRID = NP // _BLK


def _tc_matmul(x_p, W):
    def body(x_ref, w_ref, o_ref):
        o_ref[...] = jnp.dot(x_ref[...], w_ref[...],
                             preferred_element_type=jnp.float32)

    return pl.pallas_call(
        body,
        grid=(_GRID,),
        in_specs=[
            pl.BlockSpec((_BLK, D), lambda i: (i, 0)),
            pl.BlockSpec((D, D), lambda i: (0, 0)),
        ],
        out_specs=pl.BlockSpec((_BLK, D), lambda i: (i, 0)),
        out_shape=jax.ShapeDtypeStruct((NP, D), jnp.float32),
    )(x_p, W)


def _tc_scale(xw, deg2):
    def body(xw_ref, d0_ref, d1_ref, y_ref, dinv_ref):
        deg = d0_ref[0, :, 0:1] + d1_ref[0, :, 0:1] + 1.0
        dinv = lax.rsqrt(deg)
        dinv_ref[...] = dinv
        y_ref[...] = xw_ref[...] * dinv

    return pl.pallas_call(
        body,
        grid=(_GRID,),
        in_specs=[
            pl.BlockSpec((_BLK, D), lambda i: (i, 0)),
            pl.BlockSpec((1, _BLK, D), lambda i: (0, i, 0)),
            pl.BlockSpec((1, _BLK, D), lambda i: (1, i, 0)),
        ],
        out_specs=[
            pl.BlockSpec((_BLK, D), lambda i: (i, 0)),
            pl.BlockSpec((_BLK, 1), lambda i: (i, 0)),
        ],
        out_shape=[
            jax.ShapeDtypeStruct((NP, D), jnp.float32),
            jax.ShapeDtypeStruct((NP, 1), jnp.float32),
        ],
    )(xw, deg2, deg2)


def _tc_combine(acc2, y, dinv, b2):
    def body(a0_ref, a1_ref, y_ref, dinv_ref, b_ref, o_ref):
        s = (a0_ref[0] + a1_ref[0] + y_ref[...]) * dinv_ref[...] + b_ref[...]
        o_ref[...] = jnp.where(s >= 0, s, 0.1 * s)

    return pl.pallas_call(
        body,
        grid=(_GRID,),
        in_specs=[
            pl.BlockSpec((1, _BLK, D), lambda i: (0, i, 0)),
            pl.BlockSpec((1, _BLK, D), lambda i: (1, i, 0)),
            pl.BlockSpec((_BLK, D), lambda i: (i, 0)),
            pl.BlockSpec((_BLK, 1), lambda i: (i, 0)),
            pl.BlockSpec((1, D), lambda i: (0, 0)),
        ],
        out_specs=pl.BlockSpec((_BLK, D), lambda i: (i, 0)),
        out_shape=jax.ShapeDtypeStruct((NP, D), jnp.float32),
    )(acc2, acc2, y, dinv, b2)


def kernel(x, edge_index, W, b):
    row = edge_index[0]
    col = edge_index[1]
    pad_idx = N + (jnp.arange(EP - E, dtype=jnp.int32) % (NP - N))
    row_p = jnp.concatenate([row, pad_idx]).reshape(NW, CPT, CHUNK)
    col_p = jnp.concatenate([col, pad_idx]).reshape(NW, CPT, CHUNK)
    x_p = jnp.concatenate([x, jnp.zeros((NP - N, D), x.dtype)], axis=0)
    onesD = jnp.ones((CHUNK, D), jnp.float32)
    zerosD = jnp.zeros((NP, D), jnp.float32)
    b2 = b.reshape(1, D)

    deg2 = _sc_degree(col_p, zerosD, onesD)
    xw = _tc_matmul(x_p, W)
    y, dinv = _tc_scale(xw, deg2)
    acc2 = _sc_aggregate(row_p, col_p, y, zerosD)
    outp = _tc_combine(acc2, y, dinv, b2)
    return outp[:N], edge_index


# speedup vs baseline: 27.3114x; 1.1467x over previous
import functools

import jax
import jax.numpy as jnp
from jax import lax
from jax.experimental import pallas as pl
from jax.experimental.pallas import tpu as pltpu
from jax.experimental.pallas import tpu_sc as plsc

N = 10000
E = 320000
D = 128
NC = 2
NS = 16
NW = NC * NS
CHUNK = 128
CPT = 80
EP = NW * CPT * CHUNK
NP = 10240
RPS = NP // NS
ZB = RPS // CHUNK
HPT = CPT // 2

_mesh = plsc.VectorSubcoreMesh(core_axis_name="c", subcore_axis_name="s")


def _zero_shared(zblk_hbm, shared, sid):
    @pl.loop(0, ZB)
    def _(k):
        pltpu.sync_copy(zblk_hbm, shared.at[pl.ds(sid * RPS + k * CHUNK, CHUNK)])


@functools.partial(
    pl.kernel,
    mesh=_mesh,
    out_type=jax.ShapeDtypeStruct((NC, NP, D), jnp.float32),
    scratch_types=[
        pltpu.VMEM((CPT, CHUNK), jnp.int32),
        pltpu.VMEM((CHUNK, D), jnp.float32),
        pltpu.VMEM_SHARED((NP, D), jnp.float32),
    ],
)
def _sc_degree(col_hbm, zblk_hbm, ones_hbm, deg_out, idx_v, ones_v, deg_sh):
    cid = lax.axis_index("c")
    sid = lax.axis_index("s")
    wid = cid * NS + sid
    _zero_shared(zblk_hbm, deg_sh, sid)
    pltpu.sync_copy(ones_hbm, ones_v)
    pltpu.sync_copy(col_hbm.at[wid], idx_v)
    plsc.subcore_barrier()

    @pl.loop(0, CPT)
    def _(j):
        pltpu.sync_copy(ones_v, deg_sh.at[idx_v.at[j]], add=True)

    plsc.subcore_barrier()
    pltpu.sync_copy(deg_sh.at[pl.ds(sid * RPS, RPS)],
                    deg_out.at[cid].at[pl.ds(sid * RPS, RPS)])


@functools.partial(
    pl.kernel,
    mesh=_mesh,
    out_type=jax.ShapeDtypeStruct((NC, NP, D), jnp.float32),
    scratch_types=[
        pltpu.VMEM((HPT, CHUNK), jnp.int32),
        pltpu.VMEM((HPT, CHUNK), jnp.int32),
        pltpu.VMEM((CHUNK, D), jnp.float32),
        pltpu.VMEM((CHUNK, D), jnp.float32),
        pltpu.VMEM_SHARED((NP, D), jnp.float32),
        pltpu.SemaphoreType.DMA,
        pltpu.SemaphoreType.DMA,
    ],
)
def _sc_aggregate(row_hbm, col_hbm, y_hbm, zblk_hbm, acc_out,
                  ridx_v, cidx_v, gbuf0, gbuf1, acc_sh, sem0, sem1):
    cid = lax.axis_index("c")
    sid = lax.axis_index("s")
    wid = cid * NS + sid
    _zero_shared(zblk_hbm, acc_sh, sid)
    plsc.subcore_barrier()

    def start(j, buf, sem):
        pltpu.async_copy(y_hbm.at[ridx_v.at[j]], buf, sem)

    def wait(buf, sem):
        pltpu.make_async_copy(y_hbm.at[ridx_v.at[0]], buf, sem).wait()

    def scatter(j, buf):
        pltpu.sync_copy(buf, acc_sh.at[cidx_v.at[j]], add=True)

    for h in range(2):
        pltpu.sync_copy(row_hbm.at[wid].at[pl.ds(h * HPT, HPT)], ridx_v)
        pltpu.sync_copy(col_hbm.at[wid].at[pl.ds(h * HPT, HPT)], cidx_v)
        start(0, gbuf0, sem0)

        @pl.loop(0, HPT // 2 - 1)
        def _(p):
            j = 2 * p
            start(j + 1, gbuf1, sem1)
            wait(gbuf0, sem0)
            scatter(j, gbuf0)
            start(j + 2, gbuf0, sem0)
            wait(gbuf1, sem1)
            scatter(j + 1, gbuf1)

        start(HPT - 1, gbuf1, sem1)
        wait(gbuf0, sem0)
        scatter(HPT - 2, gbuf0)
        wait(gbuf1, sem1)
        scatter(HPT - 1, gbuf1)

    plsc.subcore_barrier()
    pltpu.sync_copy(acc_sh.at[pl.ds(sid * RPS, RPS)],
                    acc_out.at[cid].at[pl.ds(sid * RPS, RPS)])


_BLK = 400
_GRID = N // _BLK


def _tc_matmul(x, W):
    def body(x_ref, w_ref, o_ref):
        o_ref[...] = jnp.dot(x_ref[...], w_ref[...],
                             preferred_element_type=jnp.float32)

    return pl.pallas_call(
        body,
        grid=(_GRID,),
        in_specs=[
            pl.BlockSpec((_BLK, D), lambda i: (i, 0)),
            pl.BlockSpec((D, D), lambda i: (0, 0)),
        ],
        out_specs=pl.BlockSpec((_BLK, D), lambda i: (i, 0)),
        out_shape=jax.ShapeDtypeStruct((N, D), jnp.float32),
    )(x, W)


def _tc_scale(xw, deg2):
    def body(xw_ref, d0_ref, d1_ref, y_ref, dinv_ref):
        deg = d0_ref[0, :, 0:1] + d1_ref[0, :, 0:1] + 1.0
        dinv = lax.rsqrt(deg)
        dinv_ref[...] = dinv
        y_ref[...] = xw_ref[...] * dinv

    return pl.pallas_call(
        body,
        grid=(_GRID,),
        in_specs=[
            pl.BlockSpec((_BLK, D), lambda i: (i, 0)),
            pl.BlockSpec((1, _BLK, D), lambda i: (0, i, 0)),
            pl.BlockSpec((1, _BLK, D), lambda i: (1, i, 0)),
        ],
        out_specs=[
            pl.BlockSpec((_BLK, D), lambda i: (i, 0)),
            pl.BlockSpec((_BLK, 1), lambda i: (i, 0)),
        ],
        out_shape=[
            jax.ShapeDtypeStruct((N, D), jnp.float32),
            jax.ShapeDtypeStruct((N, 1), jnp.float32),
        ],
    )(xw, deg2, deg2)


def _tc_combine(acc2, y, dinv, b2):
    def body(a0_ref, a1_ref, y_ref, dinv_ref, b_ref, o_ref):
        s = (a0_ref[0] + a1_ref[0] + y_ref[...]) * dinv_ref[...] + b_ref[...]
        o_ref[...] = jnp.where(s >= 0, s, 0.1 * s)

    return pl.pallas_call(
        body,
        grid=(_GRID,),
        in_specs=[
            pl.BlockSpec((1, _BLK, D), lambda i: (0, i, 0)),
            pl.BlockSpec((1, _BLK, D), lambda i: (1, i, 0)),
            pl.BlockSpec((_BLK, D), lambda i: (i, 0)),
            pl.BlockSpec((_BLK, 1), lambda i: (i, 0)),
            pl.BlockSpec((1, D), lambda i: (0, 0)),
        ],
        out_specs=pl.BlockSpec((_BLK, D), lambda i: (i, 0)),
        out_shape=jax.ShapeDtypeStruct((N, D), jnp.float32),
    )(acc2, acc2, y, dinv, b2)


def kernel(x, edge_index, W, b):
    row = edge_index[0]
    col = edge_index[1]
    pad = jnp.arange(EP - E, dtype=jnp.int32)
    row_p = jnp.concatenate([row, pad % N]).reshape(NW, CPT, CHUNK)
    col_p = jnp.concatenate([col, N + pad % (NP - N)]).reshape(NW, CPT, CHUNK)
    zblk = jnp.zeros((CHUNK, D), jnp.float32)
    onesD = jnp.ones((CHUNK, D), jnp.float32)
    b2 = b.reshape(1, D)

    deg2 = _sc_degree(col_p, zblk, onesD)
    xw = _tc_matmul(x, W)
    y, dinv = _tc_scale(xw, deg2)
    acc2 = _sc_aggregate(row_p, col_p, y, zblk)
    outp = _tc_combine(acc2, y, dinv, b2)
    return outp, edge_index


# speedup vs baseline: 29.7974x; 1.0910x over previous
import functools

import jax
import jax.numpy as jnp
from jax import lax
from jax.experimental import pallas as pl
from jax.experimental.pallas import tpu as pltpu
from jax.experimental.pallas import tpu_sc as plsc

N = 10000
E = 320000
D = 128
NC = 2
NS = 16
NW = NC * NS
CHUNK = 128
CPT = 80
EP = NW * CPT * CHUNK
NP = 10240
RPS = NP // NS
ZB = RPS // CHUNK
HPT = CPT // 2

_mesh = plsc.VectorSubcoreMesh(core_axis_name="c", subcore_axis_name="s")


def _zero_shared(zeros_hbm, shared, sid):
    pltpu.sync_copy(zeros_hbm.at[pl.ds(sid * RPS, RPS)],
                    shared.at[pl.ds(sid * RPS, RPS)])


@functools.partial(
    pl.kernel,
    mesh=_mesh,
    out_type=jax.ShapeDtypeStruct((NC, NP, D), jnp.float32),
    scratch_types=[
        pltpu.VMEM((CPT, CHUNK), jnp.int32),
        pltpu.VMEM((CHUNK, D), jnp.float32),
        pltpu.VMEM_SHARED((NP, D), jnp.float32),
    ],
)
def _sc_degree(col_hbm, zeros_hbm, ones_hbm, deg_out, idx_v, ones_v, deg_sh):
    cid = lax.axis_index("c")
    sid = lax.axis_index("s")
    wid = cid * NS + sid
    _zero_shared(zeros_hbm, deg_sh, sid)
    pltpu.sync_copy(ones_hbm, ones_v)
    pltpu.sync_copy(col_hbm.at[wid], idx_v)
    plsc.subcore_barrier()

    @pl.loop(0, CPT)
    def _(j):
        pltpu.sync_copy(ones_v, deg_sh.at[idx_v.at[j]], add=True)

    plsc.subcore_barrier()
    pltpu.sync_copy(deg_sh.at[pl.ds(sid * RPS, RPS)],
                    deg_out.at[cid].at[pl.ds(sid * RPS, RPS)])


@functools.partial(
    pl.kernel,
    mesh=_mesh,
    out_type=jax.ShapeDtypeStruct((NC, NP, D), jnp.float32),
    scratch_types=[
        pltpu.VMEM((HPT, CHUNK), jnp.int32),
        pltpu.VMEM((HPT, CHUNK), jnp.int32),
        pltpu.VMEM((CHUNK, D), jnp.float32),
        pltpu.VMEM((CHUNK, D), jnp.float32),
        pltpu.VMEM_SHARED((NP, D), jnp.float32),
        pltpu.SemaphoreType.DMA,
        pltpu.SemaphoreType.DMA,
    ],
)
def _sc_aggregate(row_hbm, col_hbm, y_hbm, zeros_hbm, acc_out,
                  ridx_v, cidx_v, gbuf0, gbuf1, acc_sh, sem0, sem1):
    cid = lax.axis_index("c")
    sid = lax.axis_index("s")
    wid = cid * NS + sid
    _zero_shared(zeros_hbm, acc_sh, sid)
    plsc.subcore_barrier()

    def start(j, buf, sem):
        pltpu.async_copy(y_hbm.at[ridx_v.at[j]], buf, sem)

    def wait(buf, sem):
        pltpu.make_async_copy(y_hbm.at[ridx_v.at[0]], buf, sem).wait()

    def scatter(j, buf):
        pltpu.sync_copy(buf, acc_sh.at[cidx_v.at[j]], add=True)

    for h in range(2):
        pltpu.sync_copy(row_hbm.at[wid].at[pl.ds(h * HPT, HPT)], ridx_v)
        pltpu.sync_copy(col_hbm.at[wid].at[pl.ds(h * HPT, HPT)], cidx_v)
        start(0, gbuf0, sem0)

        @pl.loop(0, HPT // 2 - 1)
        def _(p):
            j = 2 * p
            start(j + 1, gbuf1, sem1)
            wait(gbuf0, sem0)
            scatter(j, gbuf0)
            start(j + 2, gbuf0, sem0)
            wait(gbuf1, sem1)
            scatter(j + 1, gbuf1)

        start(HPT - 1, gbuf1, sem1)
        wait(gbuf0, sem0)
        scatter(HPT - 2, gbuf0)
        wait(gbuf1, sem1)
        scatter(HPT - 1, gbuf1)

    plsc.subcore_barrier()
    pltpu.sync_copy(acc_sh.at[pl.ds(sid * RPS, RPS)],
                    acc_out.at[cid].at[pl.ds(sid * RPS, RPS)])


_BLK = 400
_GRID = N // _BLK


def _tc_matmul(x, W):
    def body(x_ref, w_ref, o_ref):
        o_ref[...] = jnp.dot(x_ref[...], w_ref[...],
                             preferred_element_type=jnp.float32)

    return pl.pallas_call(
        body,
        grid=(_GRID,),
        in_specs=[
            pl.BlockSpec((_BLK, D), lambda i: (i, 0)),
            pl.BlockSpec((D, D), lambda i: (0, 0)),
        ],
        out_specs=pl.BlockSpec((_BLK, D), lambda i: (i, 0)),
        out_shape=jax.ShapeDtypeStruct((N, D), jnp.float32),
    )(x, W)


def _tc_scale(xw, deg2):
    def body(xw_ref, d0_ref, d1_ref, y_ref, dinv_ref):
        deg = d0_ref[0, :, 0:1] + d1_ref[0, :, 0:1] + 1.0
        dinv = lax.rsqrt(deg)
        dinv_ref[...] = dinv
        y_ref[...] = xw_ref[...] * dinv

    return pl.pallas_call(
        body,
        grid=(_GRID,),
        in_specs=[
            pl.BlockSpec((_BLK, D), lambda i: (i, 0)),
            pl.BlockSpec((1, _BLK, D), lambda i: (0, i, 0)),
            pl.BlockSpec((1, _BLK, D), lambda i: (1, i, 0)),
        ],
        out_specs=[
            pl.BlockSpec((_BLK, D), lambda i: (i, 0)),
            pl.BlockSpec((_BLK, 1), lambda i: (i, 0)),
        ],
        out_shape=[
            jax.ShapeDtypeStruct((N, D), jnp.float32),
            jax.ShapeDtypeStruct((N, 1), jnp.float32),
        ],
    )(xw, deg2, deg2)


def _tc_combine(acc2, y, dinv, b2):
    def body(a0_ref, a1_ref, y_ref, dinv_ref, b_ref, o_ref):
        s = (a0_ref[0] + a1_ref[0] + y_ref[...]) * dinv_ref[...] + b_ref[...]
        o_ref[...] = jnp.where(s >= 0, s, 0.1 * s)

    return pl.pallas_call(
        body,
        grid=(_GRID,),
        in_specs=[
            pl.BlockSpec((1, _BLK, D), lambda i: (0, i, 0)),
            pl.BlockSpec((1, _BLK, D), lambda i: (1, i, 0)),
            pl.BlockSpec((_BLK, D), lambda i: (i, 0)),
            pl.BlockSpec((_BLK, 1), lambda i: (i, 0)),
            pl.BlockSpec((1, D), lambda i: (0, 0)),
        ],
        out_specs=pl.BlockSpec((_BLK, D), lambda i: (i, 0)),
        out_shape=jax.ShapeDtypeStruct((N, D), jnp.float32),
    )(acc2, acc2, y, dinv, b2)


def kernel(x, edge_index, W, b):
    row = edge_index[0]
    col = edge_index[1]
    pad = jnp.arange(EP - E, dtype=jnp.int32)
    row_p = jnp.concatenate([row, pad % N]).reshape(NW, CPT, CHUNK)
    col_p = jnp.concatenate([col, N + pad % (NP - N)]).reshape(NW, CPT, CHUNK)
    zeros = jnp.zeros((NP, D), jnp.float32)
    b2 = b.reshape(1, D)

    onesD = jnp.ones((CHUNK, D), jnp.float32)
    deg2 = _sc_degree(col_p, zeros, onesD)
    xw = _tc_matmul(x, W)
    y, dinv = _tc_scale(xw, deg2)
    acc2 = _sc_aggregate(row_p, col_p, y, zeros)
    outp = _tc_combine(acc2, y, dinv, b2)
    return outp, edge_index


# speedup vs baseline: 29.8002x; 1.0001x over previous
import functools

import jax
import jax.numpy as jnp
from jax import lax
from jax.experimental import pallas as pl
from jax.experimental.pallas import tpu as pltpu
from jax.experimental.pallas import tpu_sc as plsc

N = 10000
E = 320000
D = 128
NC = 2
NS = 16
NW = NC * NS
CHUNK = 128
CPT = 80
EP = NW * CPT * CHUNK
NP = 10240
RPS = NP // NS
ZB = RPS // CHUNK
HPT = CPT // 2

_mesh = plsc.VectorSubcoreMesh(core_axis_name="c", subcore_axis_name="s")


def _zero_shared(zeros_hbm, shared, sid):
    pltpu.sync_copy(zeros_hbm.at[pl.ds(sid * RPS, RPS)],
                    shared.at[pl.ds(sid * RPS, RPS)])


@functools.partial(
    pl.kernel,
    mesh=_mesh,
    out_type=jax.ShapeDtypeStruct((NC, NP, D), jnp.float32),
    scratch_types=[
        pltpu.VMEM((CPT, CHUNK), jnp.int32),
        pltpu.VMEM((CHUNK, D), jnp.float32),
        pltpu.VMEM_SHARED((NP, D), jnp.float32),
        pltpu.SemaphoreType.DMA,
    ],
)
def _sc_degree(col_hbm, zeros_hbm, ones_hbm, deg_out, idx_v, ones_v, deg_sh, sem):
    cid = lax.axis_index("c")
    sid = lax.axis_index("s")
    wid = cid * NS + sid
    _zero_shared(zeros_hbm, deg_sh, sid)
    pltpu.sync_copy(ones_hbm, ones_v)
    pltpu.sync_copy(col_hbm.at[wid], idx_v)
    plsc.subcore_barrier()

    @pl.loop(0, CPT)
    def _(j):
        pltpu.async_copy(ones_v, deg_sh.at[idx_v.at[j]], sem, add=True)

    @pl.loop(0, CPT)
    def _(j):
        pltpu.make_async_copy(ones_v, deg_sh.at[idx_v.at[0]], sem).wait()

    plsc.subcore_barrier()
    pltpu.sync_copy(deg_sh.at[pl.ds(sid * RPS, RPS)],
                    deg_out.at[cid].at[pl.ds(sid * RPS, RPS)])


@functools.partial(
    pl.kernel,
    mesh=_mesh,
    out_type=jax.ShapeDtypeStruct((NC, NP, D), jnp.float32),
    scratch_types=[
        pltpu.VMEM((HPT, CHUNK), jnp.int32),
        pltpu.VMEM((HPT, CHUNK), jnp.int32),
        pltpu.VMEM((CHUNK, D), jnp.float32),
        pltpu.VMEM((CHUNK, D), jnp.float32),
        pltpu.VMEM_SHARED((NP, D), jnp.float32),
        pltpu.SemaphoreType.DMA,
        pltpu.SemaphoreType.DMA,
    ],
)
def _sc_aggregate(row_hbm, col_hbm, y_hbm, zeros_hbm, acc_out,
                  ridx_v, cidx_v, gbuf0, gbuf1, acc_sh, sem0, sem1):
    cid = lax.axis_index("c")
    sid = lax.axis_index("s")
    wid = cid * NS + sid
    _zero_shared(zeros_hbm, acc_sh, sid)
    plsc.subcore_barrier()

    def start(j, buf, sem):
        pltpu.async_copy(y_hbm.at[ridx_v.at[j]], buf, sem)

    def wait(buf, sem):
        pltpu.make_async_copy(y_hbm.at[ridx_v.at[0]], buf, sem).wait()

    def scatter(j, buf):
        pltpu.sync_copy(buf, acc_sh.at[cidx_v.at[j]], add=True)

    for h in range(2):
        pltpu.sync_copy(row_hbm.at[wid].at[pl.ds(h * HPT, HPT)], ridx_v)
        pltpu.sync_copy(col_hbm.at[wid].at[pl.ds(h * HPT, HPT)], cidx_v)
        start(0, gbuf0, sem0)

        @pl.loop(0, HPT // 2 - 1)
        def _(p):
            j = 2 * p
            start(j + 1, gbuf1, sem1)
            wait(gbuf0, sem0)
            scatter(j, gbuf0)
            start(j + 2, gbuf0, sem0)
            wait(gbuf1, sem1)
            scatter(j + 1, gbuf1)

        start(HPT - 1, gbuf1, sem1)
        wait(gbuf0, sem0)
        scatter(HPT - 2, gbuf0)
        wait(gbuf1, sem1)
        scatter(HPT - 1, gbuf1)

    plsc.subcore_barrier()
    pltpu.sync_copy(acc_sh.at[pl.ds(sid * RPS, RPS)],
                    acc_out.at[cid].at[pl.ds(sid * RPS, RPS)])


_BLK = 400
_GRID = N // _BLK


def _tc_matmul(x, W):
    def body(x_ref, w_ref, o_ref):
        o_ref[...] = jnp.dot(x_ref[...], w_ref[...],
                             preferred_element_type=jnp.float32)

    return pl.pallas_call(
        body,
        grid=(_GRID,),
        in_specs=[
            pl.BlockSpec((_BLK, D), lambda i: (i, 0)),
            pl.BlockSpec((D, D), lambda i: (0, 0)),
        ],
        out_specs=pl.BlockSpec((_BLK, D), lambda i: (i, 0)),
        out_shape=jax.ShapeDtypeStruct((N, D), jnp.float32),
    )(x, W)


def _tc_scale(xw, deg2):
    def body(xw_ref, d0_ref, d1_ref, y_ref, dinv_ref):
        deg = d0_ref[0, :, 0:1] + d1_ref[0, :, 0:1] + 1.0
        dinv = lax.rsqrt(deg)
        dinv_ref[...] = dinv
        y_ref[...] = xw_ref[...] * dinv

    return pl.pallas_call(
        body,
        grid=(_GRID,),
        in_specs=[
            pl.BlockSpec((_BLK, D), lambda i: (i, 0)),
            pl.BlockSpec((1, _BLK, D), lambda i: (0, i, 0)),
            pl.BlockSpec((1, _BLK, D), lambda i: (1, i, 0)),
        ],
        out_specs=[
            pl.BlockSpec((_BLK, D), lambda i: (i, 0)),
            pl.BlockSpec((_BLK, 1), lambda i: (i, 0)),
        ],
        out_shape=[
            jax.ShapeDtypeStruct((N, D), jnp.float32),
            jax.ShapeDtypeStruct((N, 1), jnp.float32),
        ],
    )(xw, deg2, deg2)


def _tc_combine(acc2, y, dinv, b2):
    def body(a0_ref, a1_ref, y_ref, dinv_ref, b_ref, o_ref):
        s = (a0_ref[0] + a1_ref[0] + y_ref[...]) * dinv_ref[...] + b_ref[...]
        o_ref[...] = jnp.where(s >= 0, s, 0.1 * s)

    return pl.pallas_call(
        body,
        grid=(_GRID,),
        in_specs=[
            pl.BlockSpec((1, _BLK, D), lambda i: (0, i, 0)),
            pl.BlockSpec((1, _BLK, D), lambda i: (1, i, 0)),
            pl.BlockSpec((_BLK, D), lambda i: (i, 0)),
            pl.BlockSpec((_BLK, 1), lambda i: (i, 0)),
            pl.BlockSpec((1, D), lambda i: (0, 0)),
        ],
        out_specs=pl.BlockSpec((_BLK, D), lambda i: (i, 0)),
        out_shape=jax.ShapeDtypeStruct((N, D), jnp.float32),
    )(acc2, acc2, y, dinv, b2)


def kernel(x, edge_index, W, b):
    row = edge_index[0]
    col = edge_index[1]
    pad = jnp.arange(EP - E, dtype=jnp.int32)
    row_p = jnp.concatenate([row, pad % N]).reshape(NW, CPT, CHUNK)
    col_p = jnp.concatenate([col, N + pad % (NP - N)]).reshape(NW, CPT, CHUNK)
    zeros = jnp.zeros((NP, D), jnp.float32)
    b2 = b.reshape(1, D)

    onesD = jnp.ones((CHUNK, D), jnp.float32)
    deg2 = _sc_degree(col_p, zeros, onesD)
    xw = _tc_matmul(x, W)
    y, dinv = _tc_scale(xw, deg2)
    acc2 = _sc_aggregate(row_p, col_p, y, zeros)
    outp = _tc_combine(acc2, y, dinv, b2)
    return outp, edge_index


# speedup vs baseline: 38.2476x; 1.2835x over previous
import dataclasses
import functools

import jax
import jax.numpy as jnp
from jax import lax
from jax.experimental import pallas as pl
from jax.experimental.pallas import tpu as pltpu
from jax.experimental.pallas import tpu_sc as plsc

N = 10000
E = 320000
D = 128
NC = 2
NS = 16
NW = NC * NS
CHUNK = 128
CPT = 80
EP = NW * CPT * CHUNK
NP = 10240
RPS = NP // NS
ZB = RPS // CHUNK
HPT = CPT // 2

_mesh = plsc.VectorSubcoreMesh(core_axis_name="c", subcore_axis_name="s")

_cp = pltpu.CompilerParams()
if "needs_layout_passes" in pltpu.CompilerParams.__dataclass_fields__:
    _cp = dataclasses.replace(_cp, needs_layout_passes=False)


def _zero_shared(zeros_hbm, shared, sid):
    pltpu.sync_copy(zeros_hbm.at[pl.ds(sid * RPS, RPS)],
                    shared.at[pl.ds(sid * RPS, RPS)])


@functools.partial(
    pl.kernel,
    mesh=_mesh,
    out_type=jax.ShapeDtypeStruct((NC, NP), jnp.float32),
    compiler_params=_cp,
    scratch_types=[
        pltpu.VMEM((CPT, CHUNK), jnp.int32),
        pltpu.VMEM((NP,), jnp.float32),
        pltpu.VMEM((RPS,), jnp.float32),
        pltpu.VMEM((RPS,), jnp.float32),
        pltpu.VMEM_SHARED((NS, NP), jnp.float32),
    ],
)
def _sc_degree(col_hbm, deg_out, idx_v, hist_v, tmp_v, accs_v, hist_sh):
    cid = lax.axis_index("c")
    sid = lax.axis_index("s")
    wid = cid * NS + sid
    pltpu.sync_copy(col_hbm.at[wid], idx_v)
    zeros16 = jnp.zeros((16,), jnp.float32)
    ones16 = jnp.ones((16,), jnp.float32)

    @pl.loop(0, NP // 16)
    def _(k):
        hist_v[pl.ds(k * 16, 16)] = zeros16

    @pl.loop(0, CPT)
    def _(j):
        @pl.loop(0, CHUNK // 16)
        def _(k):
            idx16 = idx_v[j, pl.ds(k * 16, 16)]
            plsc.addupdate_scatter(hist_v, [idx16], ones16)

    pltpu.sync_copy(hist_v, hist_sh.at[sid])
    plsc.subcore_barrier()

    @pl.loop(0, RPS // 16)
    def _(k):
        accs_v[pl.ds(k * 16, 16)] = zeros16

    @pl.loop(0, NS)
    def _(t):
        pltpu.sync_copy(hist_sh.at[t].at[pl.ds(sid * RPS, RPS)], tmp_v)

        @pl.loop(0, RPS // 16)
        def _(k):
            slc = pl.ds(k * 16, 16)
            accs_v[slc] = accs_v[slc] + tmp_v[slc]

    pltpu.sync_copy(accs_v, deg_out.at[cid].at[pl.ds(sid * RPS, RPS)])


@functools.partial(
    pl.kernel,
    mesh=_mesh,
    out_type=jax.ShapeDtypeStruct((NC, NP, D), jnp.float32),
    scratch_types=[
        pltpu.VMEM((HPT, CHUNK), jnp.int32),
        pltpu.VMEM((HPT, CHUNK), jnp.int32),
        pltpu.VMEM((CHUNK, D), jnp.float32),
        pltpu.VMEM((CHUNK, D), jnp.float32),
        pltpu.VMEM_SHARED((NP, D), jnp.float32),
        pltpu.SemaphoreType.DMA,
        pltpu.SemaphoreType.DMA,
    ],
)
def _sc_aggregate(row_hbm, col_hbm, y_hbm, zeros_hbm, acc_out,
                  ridx_v, cidx_v, gbuf0, gbuf1, acc_sh, sem0, sem1):
    cid = lax.axis_index("c")
    sid = lax.axis_index("s")
    wid = cid * NS + sid
    _zero_shared(zeros_hbm, acc_sh, sid)
    plsc.subcore_barrier()

    def start(j, buf, sem):
        pltpu.async_copy(y_hbm.at[ridx_v.at[j]], buf, sem)

    def wait(buf, sem):
        pltpu.make_async_copy(y_hbm.at[ridx_v.at[0]], buf, sem).wait()

    def scatter(j, buf):
        pltpu.sync_copy(buf, acc_sh.at[cidx_v.at[j]], add=True)

    for h in range(2):
        pltpu.sync_copy(row_hbm.at[wid].at[pl.ds(h * HPT, HPT)], ridx_v)
        pltpu.sync_copy(col_hbm.at[wid].at[pl.ds(h * HPT, HPT)], cidx_v)
        start(0, gbuf0, sem0)

        @pl.loop(0, HPT // 2 - 1)
        def _(p):
            j = 2 * p
            start(j + 1, gbuf1, sem1)
            wait(gbuf0, sem0)
            scatter(j, gbuf0)
            start(j + 2, gbuf0, sem0)
            wait(gbuf1, sem1)
            scatter(j + 1, gbuf1)

        start(HPT - 1, gbuf1, sem1)
        wait(gbuf0, sem0)
        scatter(HPT - 2, gbuf0)
        wait(gbuf1, sem1)
        scatter(HPT - 1, gbuf1)

    plsc.subcore_barrier()
    pltpu.sync_copy(acc_sh.at[pl.ds(sid * RPS, RPS)],
                    acc_out.at[cid].at[pl.ds(sid * RPS, RPS)])


_BLK = 400
_GRID = N // _BLK


def _tc_matmul(x, W):
    def body(x_ref, w_ref, o_ref):
        o_ref[...] = jnp.dot(x_ref[...], w_ref[...],
                             preferred_element_type=jnp.float32)

    return pl.pallas_call(
        body,
        grid=(_GRID,),
        in_specs=[
            pl.BlockSpec((_BLK, D), lambda i: (i, 0)),
            pl.BlockSpec((D, D), lambda i: (0, 0)),
        ],
        out_specs=pl.BlockSpec((_BLK, D), lambda i: (i, 0)),
        out_shape=jax.ShapeDtypeStruct((N, D), jnp.float32),
    )(x, W)


def _tc_scale(xw, deg2):
    def body(xw_ref, d0_ref, d1_ref, y_ref, dinv_ref):
        deg = d0_ref[0, 0, 0][:, None] + d1_ref[0, 0, 0][:, None] + 1.0
        dinv = lax.rsqrt(deg)
        dinv_ref[...] = dinv
        y_ref[...] = xw_ref[...] * dinv

    return pl.pallas_call(
        body,
        grid=(_GRID,),
        in_specs=[
            pl.BlockSpec((_BLK, D), lambda i: (i, 0)),
            pl.BlockSpec((1, 1, 1, _BLK), lambda i: (0, i, 0, 0)),
            pl.BlockSpec((1, 1, 1, _BLK), lambda i: (1, i, 0, 0)),
        ],
        out_specs=[
            pl.BlockSpec((_BLK, D), lambda i: (i, 0)),
            pl.BlockSpec((_BLK, 1), lambda i: (i, 0)),
        ],
        out_shape=[
            jax.ShapeDtypeStruct((N, D), jnp.float32),
            jax.ShapeDtypeStruct((N, 1), jnp.float32),
        ],
    )(xw, deg2, deg2)


def _tc_combine(acc2, y, dinv, b2):
    def body(a0_ref, a1_ref, y_ref, dinv_ref, b_ref, o_ref):
        s = (a0_ref[0] + a1_ref[0] + y_ref[...]) * dinv_ref[...] + b_ref[...]
        o_ref[...] = jnp.where(s >= 0, s, 0.1 * s)

    return pl.pallas_call(
        body,
        grid=(_GRID,),
        in_specs=[
            pl.BlockSpec((1, _BLK, D), lambda i: (0, i, 0)),
            pl.BlockSpec((1, _BLK, D), lambda i: (1, i, 0)),
            pl.BlockSpec((_BLK, D), lambda i: (i, 0)),
            pl.BlockSpec((_BLK, 1), lambda i: (i, 0)),
            pl.BlockSpec((1, D), lambda i: (0, 0)),
        ],
        out_specs=pl.BlockSpec((_BLK, D), lambda i: (i, 0)),
        out_shape=jax.ShapeDtypeStruct((N, D), jnp.float32),
    )(acc2, acc2, y, dinv, b2)


def kernel(x, edge_index, W, b):
    row = edge_index[0]
    col = edge_index[1]
    pad = jnp.arange(EP - E, dtype=jnp.int32)
    row_p = jnp.concatenate([row, pad % N]).reshape(NW, CPT, CHUNK)
    col_p = jnp.concatenate([col, N + pad % (NP - N)]).reshape(NW, CPT, CHUNK)
    zeros = jnp.zeros((NP, D), jnp.float32)
    b2 = b.reshape(1, D)

    deg2 = _sc_degree(col_p)
    xw = _tc_matmul(x, W)
    y, dinv = _tc_scale(xw, deg2[:, :N].reshape(NC, _GRID, 1, _BLK))
    acc2 = _sc_aggregate(row_p, col_p, y, zeros)
    outp = _tc_combine(acc2, y, dinv, b2)
    return outp, edge_index


# speedup vs baseline: 38.6604x; 1.0108x over previous
import dataclasses
import functools

import jax
import jax.numpy as jnp
from jax import lax
from jax.experimental import pallas as pl
from jax.experimental.pallas import tpu as pltpu
from jax.experimental.pallas import tpu_sc as plsc

N = 10000
E = 320000
D = 128
NC = 2
NS = 16
NW = NC * NS
CHUNK = 128
CPT = 80
EP = NW * CPT * CHUNK
NP = 10240
RPS = NP // NS
ZB = RPS // CHUNK
HPT = CPT // 2

_mesh = plsc.VectorSubcoreMesh(core_axis_name="c", subcore_axis_name="s")

_cp = pltpu.CompilerParams()
if "needs_layout_passes" in pltpu.CompilerParams.__dataclass_fields__:
    _cp = dataclasses.replace(_cp, needs_layout_passes=False)


def _zero_shared(zeros_hbm, shared, sid):
    pltpu.sync_copy(zeros_hbm.at[pl.ds(sid * RPS, RPS)],
                    shared.at[pl.ds(sid * RPS, RPS)])


@functools.partial(
    pl.kernel,
    mesh=_mesh,
    out_type=jax.ShapeDtypeStruct((NC, NP), jnp.float32),
    compiler_params=_cp,
    scratch_types=[
        pltpu.VMEM((CPT, CHUNK), jnp.int32),
        pltpu.VMEM((NP,), jnp.float32),
        pltpu.VMEM((RPS,), jnp.float32),
        pltpu.VMEM((RPS,), jnp.float32),
        pltpu.VMEM_SHARED((NS, NP), jnp.float32),
    ],
)
def _sc_degree(col_hbm, deg_out, idx_v, hist_v, tmp_v, accs_v, hist_sh):
    cid = lax.axis_index("c")
    sid = lax.axis_index("s")
    wid = cid * NS + sid
    pltpu.sync_copy(col_hbm.at[wid], idx_v)
    zeros16 = jnp.zeros((16,), jnp.float32)
    ones16 = jnp.ones((16,), jnp.float32)

    @pl.loop(0, NP // 16)
    def _(k):
        hist_v[pl.ds(k * 16, 16)] = zeros16

    @pl.loop(0, CPT)
    def _(j):
        @pl.loop(0, CHUNK // 16)
        def _(k):
            idx16 = idx_v[j, pl.ds(k * 16, 16)]
            plsc.addupdate_scatter(hist_v, [idx16], ones16)

    pltpu.sync_copy(hist_v, hist_sh.at[sid])
    plsc.subcore_barrier()

    @pl.loop(0, RPS // 16)
    def _(k):
        accs_v[pl.ds(k * 16, 16)] = zeros16

    @pl.loop(0, NS)
    def _(t):
        pltpu.sync_copy(hist_sh.at[t].at[pl.ds(sid * RPS, RPS)], tmp_v)

        @pl.loop(0, RPS // 16)
        def _(k):
            slc = pl.ds(k * 16, 16)
            accs_v[slc] = accs_v[slc] + tmp_v[slc]

    pltpu.sync_copy(accs_v, deg_out.at[cid].at[pl.ds(sid * RPS, RPS)])


@functools.partial(
    pl.kernel,
    mesh=_mesh,
    out_type=jax.ShapeDtypeStruct((NC, NP, D), jnp.float32),
    scratch_types=[
        pltpu.VMEM((HPT, CHUNK), jnp.int32),
        pltpu.VMEM((HPT, CHUNK), jnp.int32),
        pltpu.VMEM((CHUNK, D), jnp.float32),
        pltpu.VMEM((CHUNK, D), jnp.float32),
        pltpu.VMEM_SHARED((NP, D), jnp.float32),
        pltpu.SemaphoreType.DMA,
        pltpu.SemaphoreType.DMA,
    ],
)
def _sc_aggregate(row_hbm, col_hbm, y_hbm, zeros_hbm, acc_out,
                  ridx_v, cidx_v, gbuf0, gbuf1, acc_sh, sem0, sem1):
    cid = lax.axis_index("c")
    sid = lax.axis_index("s")
    wid = cid * NS + sid
    _zero_shared(zeros_hbm, acc_sh, sid)
    plsc.subcore_barrier()

    def start(j, buf, sem):
        pltpu.async_copy(y_hbm.at[ridx_v.at[j]], buf, sem)

    def wait(buf, sem):
        pltpu.make_async_copy(y_hbm.at[ridx_v.at[0]], buf, sem).wait()

    def scatter(j, buf):
        pltpu.sync_copy(buf, acc_sh.at[cidx_v.at[j]], add=True)

    for h in range(2):
        pltpu.sync_copy(row_hbm.at[wid].at[pl.ds(h * HPT, HPT)], ridx_v)
        pltpu.sync_copy(col_hbm.at[wid].at[pl.ds(h * HPT, HPT)], cidx_v)
        start(0, gbuf0, sem0)

        @pl.loop(0, HPT // 2 - 1)
        def _(p):
            j = 2 * p
            start(j + 1, gbuf1, sem1)
            wait(gbuf0, sem0)
            scatter(j, gbuf0)
            start(j + 2, gbuf0, sem0)
            wait(gbuf1, sem1)
            scatter(j + 1, gbuf1)

        start(HPT - 1, gbuf1, sem1)
        wait(gbuf0, sem0)
        scatter(HPT - 2, gbuf0)
        wait(gbuf1, sem1)
        scatter(HPT - 1, gbuf1)

    plsc.subcore_barrier()
    pltpu.sync_copy(acc_sh.at[pl.ds(sid * RPS, RPS)],
                    acc_out.at[cid].at[pl.ds(sid * RPS, RPS)])


_BLK = 400
_GRID = N // _BLK


def _tc_mmscale(x, deg2, W):
    def body(x_ref, d0_ref, d1_ref, w_ref, y_ref, dinv_ref):
        deg = d0_ref[0, 0, 0][:, None] + d1_ref[0, 0, 0][:, None] + 1.0
        dinv = lax.rsqrt(deg)
        dinv_ref[...] = dinv
        y_ref[...] = jnp.dot(x_ref[...] * dinv, w_ref[...],
                             preferred_element_type=jnp.float32)

    return pl.pallas_call(
        body,
        grid=(_GRID,),
        in_specs=[
            pl.BlockSpec((_BLK, D), lambda i: (i, 0)),
            pl.BlockSpec((1, 1, 1, _BLK), lambda i: (0, i, 0, 0)),
            pl.BlockSpec((1, 1, 1, _BLK), lambda i: (1, i, 0, 0)),
            pl.BlockSpec((D, D), lambda i: (0, 0)),
        ],
        out_specs=[
            pl.BlockSpec((_BLK, D), lambda i: (i, 0)),
            pl.BlockSpec((_BLK, 1), lambda i: (i, 0)),
        ],
        out_shape=[
            jax.ShapeDtypeStruct((N, D), jnp.float32),
            jax.ShapeDtypeStruct((N, 1), jnp.float32),
        ],
    )(x, deg2, deg2, W)


def _tc_combine(acc2, y, dinv, b2):
    def body(a0_ref, a1_ref, y_ref, dinv_ref, b_ref, o_ref):
        s = (a0_ref[0] + a1_ref[0] + y_ref[...]) * dinv_ref[...] + b_ref[...]
        o_ref[...] = jnp.where(s >= 0, s, 0.1 * s)

    return pl.pallas_call(
        body,
        grid=(_GRID,),
        in_specs=[
            pl.BlockSpec((1, _BLK, D), lambda i: (0, i, 0)),
            pl.BlockSpec((1, _BLK, D), lambda i: (1, i, 0)),
            pl.BlockSpec((_BLK, D), lambda i: (i, 0)),
            pl.BlockSpec((_BLK, 1), lambda i: (i, 0)),
            pl.BlockSpec((1, D), lambda i: (0, 0)),
        ],
        out_specs=pl.BlockSpec((_BLK, D), lambda i: (i, 0)),
        out_shape=jax.ShapeDtypeStruct((N, D), jnp.float32),
    )(acc2, acc2, y, dinv, b2)


def kernel(x, edge_index, W, b):
    row = edge_index[0]
    col = edge_index[1]
    pad = jnp.arange(EP - E, dtype=jnp.int32)
    row_p = jnp.concatenate([row, pad % N]).reshape(NW, CPT, CHUNK)
    col_p = jnp.concatenate([col, N + pad % (NP - N)]).reshape(NW, CPT, CHUNK)
    zeros = jnp.zeros((NP, D), jnp.float32)
    b2 = b.reshape(1, D)

    deg2 = _sc_degree(col_p)
    y, dinv = _tc_mmscale(x, deg2[:, :N].reshape(NC, _GRID, 1, _BLK), W)
    acc2 = _sc_aggregate(row_p, col_p, y, zeros)
    outp = _tc_combine(acc2, y, dinv, b2)
    return outp, edge_index
